# Initial kernel scaffold; baseline (speedup 1.0000x reference)
import jax
import jax.numpy as jnp
from jax.experimental import pallas as pl


def kernel(points, features, W1, b1, W2, b2):
    raise NotImplementedError("write your pallas kernel here")


# speedup vs baseline: 12.5764x; 12.5764x over previous
import functools

import jax
import jax.numpy as jnp
from jax import lax
from jax.experimental import pallas as pl
from jax.experimental.pallas import tpu as pltpu
from jax.experimental.pallas import tpu_sc as plsc

_K = 16
_ROWS = 256
_PROJ_ROWS = 1024
_GCHUNK = 128
_NC, _NS = 2, 16


def _proj_body(feat_ref, pts_ref, w1cT_ref, w1nT_ref, w1eT_ref, b1_ref,
               a_ref, g_ref):
    f = feat_ref[...]
    p = pts_ref[...]
    pc = jnp.dot(f, w1cT_ref[...], precision=lax.Precision.HIGHEST)
    pn = jnp.dot(f, w1nT_ref[...], precision=lax.Precision.HIGHEST)
    pe = jnp.dot(p, w1eT_ref[...], precision=lax.Precision.HIGHEST)
    a_ref[...] = pc - pe + b1_ref[...]
    g_ref[...] = pn + pe


def _proj_call(feat, pts_pad, w1cT, w1nT, w1eT, b1row):
    BN, C = feat.shape
    H = w1cT.shape[1]
    grid = (BN // _PROJ_ROWS,)
    return pl.pallas_call(
        _proj_body,
        grid=grid,
        in_specs=[
            pl.BlockSpec((_PROJ_ROWS, C), lambda i: (i, 0)),
            pl.BlockSpec((_PROJ_ROWS, 8), lambda i: (i, 0)),
            pl.BlockSpec((C, H), lambda i: (0, 0)),
            pl.BlockSpec((C, H), lambda i: (0, 0)),
            pl.BlockSpec((8, H), lambda i: (0, 0)),
            pl.BlockSpec((1, H), lambda i: (0, 0)),
        ],
        out_specs=[
            pl.BlockSpec((_PROJ_ROWS, H), lambda i: (i, 0)),
            pl.BlockSpec((_PROJ_ROWS, H), lambda i: (i, 0)),
        ],
        out_shape=[
            jax.ShapeDtypeStruct((BN, H), jnp.float32),
            jax.ShapeDtypeStruct((BN, H), jnp.float32),
        ],
    )(feat, pts_pad, w1cT, w1nT, w1eT, b1row)


def _knn_body(q_ref, ptsT_ref, idx_ref):
    b = pl.program_id(0)
    q = q_ref[0]
    kT = ptsT_ref[0]
    dot = jnp.dot(q, kT, precision=lax.Precision.HIGHEST)
    sq_c = jnp.sum(kT * kT, axis=0, keepdims=True)
    sq_r = jnp.sum(q * q, axis=1, keepdims=True)
    vals = sq_r + sq_c - 2.0 * dot
    R, N = vals.shape
    col = lax.broadcasted_iota(jnp.int32, (R, N), 1)
    cols = []
    for _ in range(_K):
        m = jnp.min(vals, axis=1, keepdims=True)
        i = jnp.min(jnp.where(vals <= m, col, N), axis=1, keepdims=True)
        cols.append(i)
        vals = jnp.where(col == i, jnp.float32(jnp.inf), vals)
    idx_ref[0] = jnp.concatenate(cols, axis=1) + b * N


def _knn_call(pts_pad, ptsT):
    B, N, _ = pts_pad.shape
    grid = (B, N // _ROWS)
    return pl.pallas_call(
        _knn_body,
        grid=grid,
        in_specs=[
            pl.BlockSpec((1, _ROWS, 8), lambda b, i: (b, i, 0)),
            pl.BlockSpec((1, 8, N), lambda b, i: (b, 0, 0)),
        ],
        out_specs=pl.BlockSpec((1, _ROWS, _K), lambda b, i: (b, i, 0)),
        out_shape=jax.ShapeDtypeStruct((B, N, _K), jnp.int32),
    )(pts_pad, ptsT)


def _sc_gather_call(table, idx):
    M = idx.shape[0]
    D = table.shape[1]
    nw = _NC * _NS
    per_w = M // nw
    nchunk = per_w // _GCHUNK
    mesh = plsc.VectorSubcoreMesh(core_axis_name="c", subcore_axis_name="s")

    @functools.partial(
        pl.kernel, mesh=mesh,
        out_type=jax.ShapeDtypeStruct((M, D), jnp.float32),
        compiler_params=pltpu.CompilerParams(use_tc_tiling_on_sc=False),
        scratch_types=[
            pltpu.VMEM((_GCHUNK,), jnp.int32),
            pltpu.VMEM((_GCHUNK, D), jnp.float32),
            pltpu.SemaphoreType.DMA,
        ],
    )
    def gather_kernel(table_hbm, idx_hbm, out_hbm, idx_v, rows_v, sem):
        wid = lax.axis_index("s") * _NC + lax.axis_index("c")
        base = wid * per_w

        def body(c, carry):
            off = pl.multiple_of(base + c * _GCHUNK, _GCHUNK)
            pltpu.sync_copy(idx_hbm.at[pl.ds(off, _GCHUNK)], idx_v)
            pltpu.async_copy(table_hbm.at[idx_v], rows_v, sem).wait()
            pltpu.sync_copy(rows_v, out_hbm.at[pl.ds(off, _GCHUNK)])
            return carry

        lax.fori_loop(0, nchunk, body, 0)

    return gather_kernel(table, idx)


def _mlp_body(gath_ref, a_ref, w2T_ref, b2_ref, out_ref):
    R = a_ref.shape[0]
    H = a_ref.shape[1]
    gath = gath_ref[...]
    a = a_ref[...]
    h = jnp.maximum(gath.reshape(R, _K, H) + a[:, None, :], 0.0)
    ef = jnp.dot(h.reshape(R * _K, H), w2T_ref[...],
                 precision=lax.Precision.HIGHEST)
    ef = ef.reshape(R, _K, ef.shape[-1])
    out_ref[...] = jnp.max(ef, axis=1) + b2_ref[...]


def _mlp_call(gath, a, w2T, b2row):
    BN, H = a.shape
    CO = w2T.shape[1]
    grid = (BN // _ROWS,)
    return pl.pallas_call(
        _mlp_body,
        grid=grid,
        in_specs=[
            pl.BlockSpec((_ROWS * _K, H), lambda i: (i, 0)),
            pl.BlockSpec((_ROWS, H), lambda i: (i, 0)),
            pl.BlockSpec((H, CO), lambda i: (0, 0)),
            pl.BlockSpec((1, CO), lambda i: (0, 0)),
        ],
        out_specs=pl.BlockSpec((_ROWS, CO), lambda i: (i, 0)),
        out_shape=jax.ShapeDtypeStruct((BN, CO), jnp.float32),
    )(gath, a, w2T, b2row)


def kernel(points, features, W1, b1, W2, b2):
    B, N, _ = points.shape
    C = features.shape[-1]
    H = W1.shape[0]
    CO = W2.shape[0]
    BN = B * N

    pts_pad = jnp.concatenate(
        [points, jnp.zeros((B, N, 5), points.dtype)], axis=-1)
    ptsT = jnp.swapaxes(pts_pad, 1, 2)
    w1cT = jnp.transpose(W1[:, :C])
    w1nT = jnp.transpose(W1[:, C:2 * C])
    w1eT = jnp.transpose(jnp.concatenate(
        [W1[:, 2 * C:], jnp.zeros((H, 5), W1.dtype)], axis=1))
    w2T = jnp.transpose(W2)

    a, g = _proj_call(features.reshape(BN, C), pts_pad.reshape(BN, 8),
                      w1cT, w1nT, w1eT, b1.reshape(1, H))
    idx = _knn_call(pts_pad, ptsT)
    gath = _sc_gather_call(g, idx.reshape(BN * _K))
    out = _mlp_call(gath, a, w2T, b2.reshape(1, CO))
    return out.reshape(B, N, CO)


# speedup vs baseline: 20.2096x; 1.6070x over previous
import functools

import jax
import jax.numpy as jnp
from jax import lax
from jax.experimental import pallas as pl
from jax.experimental.pallas import tpu as pltpu
from jax.experimental.pallas import tpu_sc as plsc

_K = 16
_ROWS = 256
_PROJ_ROWS = 1024
_GCHUNK = 128
_NC, _NS = 2, 16


def _proj_body(feat_ref, pts_ref, w1cT_ref, w1nT_ref, w1eT_ref, b1_ref,
               a_ref, g_ref):
    f = feat_ref[...]
    p = pts_ref[...]
    pc = jnp.dot(f, w1cT_ref[...], precision=lax.Precision.HIGHEST)
    pn = jnp.dot(f, w1nT_ref[...], precision=lax.Precision.HIGHEST)
    pe = jnp.dot(p, w1eT_ref[...], precision=lax.Precision.HIGHEST)
    a_ref[...] = pc - pe + b1_ref[...]
    g_ref[...] = pn + pe


def _proj_call(feat, pts_pad, w1cT, w1nT, w1eT, b1row):
    BN, C = feat.shape
    H = w1cT.shape[1]
    grid = (BN // _PROJ_ROWS,)
    return pl.pallas_call(
        _proj_body,
        grid=grid,
        in_specs=[
            pl.BlockSpec((_PROJ_ROWS, C), lambda i: (i, 0)),
            pl.BlockSpec((_PROJ_ROWS, 8), lambda i: (i, 0)),
            pl.BlockSpec((C, H), lambda i: (0, 0)),
            pl.BlockSpec((C, H), lambda i: (0, 0)),
            pl.BlockSpec((8, H), lambda i: (0, 0)),
            pl.BlockSpec((1, H), lambda i: (0, 0)),
        ],
        out_specs=[
            pl.BlockSpec((_PROJ_ROWS, H), lambda i: (i, 0)),
            pl.BlockSpec((_PROJ_ROWS, H), lambda i: (i, 0)),
        ],
        out_shape=[
            jax.ShapeDtypeStruct((BN, H), jnp.float32),
            jax.ShapeDtypeStruct((BN, H), jnp.float32),
        ],
    )(feat, pts_pad, w1cT, w1nT, w1eT, b1row)


_DEPTH = 5
_LANES = 128


def _knn_body(q_ref, ptsT_ref, idx_ref):
    b = pl.program_id(0)
    q = q_ref[0]
    kT = ptsT_ref[0]
    dot = jnp.dot(q, kT, precision=lax.Precision.HIGHEST)
    sq_c = jnp.sum(kT * kT, axis=0, keepdims=True)
    sq_r = jnp.sum(q * q, axis=1, keepdims=True)
    d2 = sq_r + sq_c - 2.0 * dot
    R, N = d2.shape
    ngrp = N // _LANES
    inf = jnp.float32(jnp.inf)

    sv = [jnp.full((R, _LANES), inf, jnp.float32) for _ in range(_DEPTH)]
    sg = [jnp.zeros((R, _LANES), jnp.int32) for _ in range(_DEPTH)]
    for v in range(ngrp):
        x = d2[:, v * _LANES:(v + 1) * _LANES]
        xid = jnp.full((R, _LANES), v, jnp.int32)
        for l in range(_DEPTH):
            pred = x < sv[l]
            ns = jnp.minimum(sv[l], x)
            nid = jnp.where(pred, xid, sg[l])
            if l + 1 < _DEPTH:
                nx = jnp.maximum(sv[l], x)
                nxid = jnp.where(pred, sg[l], xid)
                x, xid = nx, nxid
            sv[l], sg[l] = ns, nid

    lane = lax.broadcasted_iota(jnp.int32, (R, _LANES), 1)
    cm, cg = sv[0], sg[0]
    dep = jnp.zeros((R, _LANES), jnp.int32)
    overflow = jnp.zeros((R, 1), jnp.bool_)
    tails_v = sv[1:] + [jnp.full((R, _LANES), inf, jnp.float32)]
    tails_g = sg[1:] + [jnp.zeros((R, _LANES), jnp.int32)]
    cols = []
    for _ in range(_K):
        m = jnp.min(cm, axis=1, keepdims=True)
        jstar = jnp.min(jnp.where(cm <= m, lane, _LANES), axis=1,
                        keepdims=True)
        sel = lane == jstar
        overflow = overflow | jnp.any(sel & (dep == _DEPTH - 1), axis=1,
                                      keepdims=True)
        gstar = jnp.min(jnp.where(sel, cg, ngrp), axis=1, keepdims=True)
        cols.append(gstar * _LANES + jstar)
        nv = tails_v[-1]
        ng = tails_g[-1]
        for t in range(len(tails_v) - 2, -1, -1):
            is_t = dep == t
            nv = jnp.where(is_t, tails_v[t], nv)
            ng = jnp.where(is_t, tails_g[t], ng)
        cm = jnp.where(sel, nv, cm)
        cg = jnp.where(sel, ng, cg)
        dep = jnp.where(sel, jnp.minimum(dep + 1, _DEPTH - 1), dep)
    idx_ref[0] = jnp.concatenate(cols, axis=1) + b * N

    @pl.when(jnp.any(overflow))
    def _slow():
        col = lax.broadcasted_iota(jnp.int32, (R, N), 1)
        vals = d2
        scols = []
        for _ in range(_K):
            mm = jnp.min(vals, axis=1, keepdims=True)
            i = jnp.min(jnp.where(vals <= mm, col, N), axis=1, keepdims=True)
            scols.append(i)
            vals = jnp.where(col == i, inf, vals)
        idx_ref[0] = jnp.concatenate(scols, axis=1) + b * N


def _knn_call(pts_pad, ptsT):
    B, N, _ = pts_pad.shape
    grid = (B, N // _ROWS)
    return pl.pallas_call(
        _knn_body,
        grid=grid,
        in_specs=[
            pl.BlockSpec((1, _ROWS, 8), lambda b, i: (b, i, 0)),
            pl.BlockSpec((1, 8, N), lambda b, i: (b, 0, 0)),
        ],
        out_specs=pl.BlockSpec((1, _ROWS, _K), lambda b, i: (b, i, 0)),
        out_shape=jax.ShapeDtypeStruct((B, N, _K), jnp.int32),
    )(pts_pad, ptsT)


def _sc_gather_call(table, idx):
    M = idx.shape[0]
    D = table.shape[1]
    nw = _NC * _NS
    per_w = M // nw
    nchunk = per_w // _GCHUNK
    mesh = plsc.VectorSubcoreMesh(core_axis_name="c", subcore_axis_name="s")

    @functools.partial(
        pl.kernel, mesh=mesh,
        out_type=jax.ShapeDtypeStruct((M, D), jnp.float32),
        compiler_params=pltpu.CompilerParams(use_tc_tiling_on_sc=False),
        scratch_types=[
            pltpu.VMEM((_GCHUNK,), jnp.int32),
            pltpu.VMEM((_GCHUNK, D), jnp.float32),
            pltpu.SemaphoreType.DMA,
        ],
    )
    def gather_kernel(table_hbm, idx_hbm, out_hbm, idx_v, rows_v, sem):
        wid = lax.axis_index("s") * _NC + lax.axis_index("c")
        base = wid * per_w

        def body(c, carry):
            off = pl.multiple_of(base + c * _GCHUNK, _GCHUNK)
            pltpu.sync_copy(idx_hbm.at[pl.ds(off, _GCHUNK)], idx_v)
            pltpu.async_copy(table_hbm.at[idx_v], rows_v, sem).wait()
            pltpu.sync_copy(rows_v, out_hbm.at[pl.ds(off, _GCHUNK)])
            return carry

        lax.fori_loop(0, nchunk, body, 0)

    return gather_kernel(table, idx)


def _mlp_body(gath_ref, a_ref, w2T_ref, b2_ref, out_ref):
    R = a_ref.shape[0]
    H = a_ref.shape[1]
    gath = gath_ref[...]
    a = a_ref[...]
    h = jnp.maximum(gath.reshape(R, _K, H) + a[:, None, :], 0.0)
    ef = jnp.dot(h.reshape(R * _K, H), w2T_ref[...],
                 precision=lax.Precision.HIGHEST)
    ef = ef.reshape(R, _K, ef.shape[-1])
    out_ref[...] = jnp.max(ef, axis=1) + b2_ref[...]


def _mlp_call(gath, a, w2T, b2row):
    BN, H = a.shape
    CO = w2T.shape[1]
    grid = (BN // _ROWS,)
    return pl.pallas_call(
        _mlp_body,
        grid=grid,
        in_specs=[
            pl.BlockSpec((_ROWS * _K, H), lambda i: (i, 0)),
            pl.BlockSpec((_ROWS, H), lambda i: (i, 0)),
            pl.BlockSpec((H, CO), lambda i: (0, 0)),
            pl.BlockSpec((1, CO), lambda i: (0, 0)),
        ],
        out_specs=pl.BlockSpec((_ROWS, CO), lambda i: (i, 0)),
        out_shape=jax.ShapeDtypeStruct((BN, CO), jnp.float32),
    )(gath, a, w2T, b2row)


def kernel(points, features, W1, b1, W2, b2):
    B, N, _ = points.shape
    C = features.shape[-1]
    H = W1.shape[0]
    CO = W2.shape[0]
    BN = B * N

    pts_pad = jnp.concatenate(
        [points, jnp.zeros((B, N, 5), points.dtype)], axis=-1)
    ptsT = jnp.swapaxes(pts_pad, 1, 2)
    w1cT = jnp.transpose(W1[:, :C])
    w1nT = jnp.transpose(W1[:, C:2 * C])
    w1eT = jnp.transpose(jnp.concatenate(
        [W1[:, 2 * C:], jnp.zeros((H, 5), W1.dtype)], axis=1))
    w2T = jnp.transpose(W2)

    a, g = _proj_call(features.reshape(BN, C), pts_pad.reshape(BN, 8),
                      w1cT, w1nT, w1eT, b1.reshape(1, H))
    idx = _knn_call(pts_pad, ptsT)
    gath = _sc_gather_call(g, idx.reshape(BN * _K))
    out = _mlp_call(gath, a, w2T, b2.reshape(1, CO))
    return out.reshape(B, N, CO)
